# baseline (speedup 1.0000x reference)
import functools

import jax
import jax.numpy as jnp
from jax import lax
from jax.experimental import pallas as pl
from jax.experimental.pallas import tpu as pltpu
from jax.experimental.pallas import tpu_sc as plsc

EMBED_DIM = 32
BATCH = 16384
NUM_ROWS = 100001
HALF = 16
SUB = 2
TAB_ROWS = NUM_ROWS * SUB
NUM_CORES = 2
NUM_SUBCORES = 16
NUM_WORKERS = NUM_CORES * NUM_SUBCORES
B_PER_W = BATCH // NUM_WORKERS
CHUNK = 128
N_CHUNKS = B_PER_W // CHUNK
LANES = 16
N_GROUPS = B_PER_W // LANES

_mesh = plsc.VectorSubcoreMesh(core_axis_name="c", subcore_axis_name="s")


@functools.partial(
    pl.kernel,
    mesh=_mesh,
    compiler_params=pltpu.CompilerParams(use_tc_tiling_on_sc=False),
    out_type=jax.ShapeDtypeStruct(
        (SUB, NUM_WORKERS, N_CHUNKS, CHUNK, HALF), jnp.float32
    ),
    scratch_types=[
        pltpu.VMEM((B_PER_W,), jnp.int32),
        pltpu.VMEM((B_PER_W,), jnp.int32),
        pltpu.VMEM((B_PER_W,), jnp.int32),
        pltpu.VMEM((N_CHUNKS, CHUNK, HALF), jnp.float32),
        pltpu.VMEM((N_CHUNKS, CHUNK, HALF), jnp.float32),
        pltpu.SemaphoreType.DMA,
    ],
)
def _gather_kernel(
    tab_hbm, idx_hbm, out_hbm, idx_v, eflat, oflat, erows, orows, sem
):
    wid = lax.axis_index("s") * NUM_CORES + lax.axis_index("c")
    base = wid * B_PER_W
    pltpu.sync_copy(idx_hbm.at[pl.ds(base, B_PER_W)], idx_v)
    for g in range(N_GROUPS):
        v = idx_v[pl.ds(g * LANES, LANES)]
        e = v * 2
        eflat[pl.ds(g * LANES, LANES)] = e
        oflat[pl.ds(g * LANES, LANES)] = e + 1
    copies = []
    for j in range(N_CHUNKS):
        copies.append(
            pltpu.async_copy(
                tab_hbm.at[eflat.at[pl.ds(j * CHUNK, CHUNK)]], erows.at[j], sem
            )
        )
        copies.append(
            pltpu.async_copy(
                tab_hbm.at[oflat.at[pl.ds(j * CHUNK, CHUNK)]], orows.at[j], sem
            )
        )
    for c in copies:
        c.wait()
    pltpu.sync_copy(erows, out_hbm.at[0, wid])
    pltpu.sync_copy(orows, out_hbm.at[1, wid])


def kernel(indices, table):
    tab2 = table.reshape(TAB_ROWS, HALF)
    idx = indices.astype(jnp.int32)
    out = _gather_kernel(tab2, idx)
    return jnp.transpose(out, (1, 2, 3, 0, 4)).reshape(BATCH, EMBED_DIM)


# speedup vs baseline: 1.0268x; 1.0268x over previous
import functools
import jax
import jax.numpy as jnp
from jax import lax
from jax.experimental import pallas as pl
from jax.experimental.pallas import tpu as pltpu
from jax.experimental.pallas import tpu_sc as plsc

EMBED_DIM = 32
BATCH = 16384
NUM_ROWS = 100001
FLAT = NUM_ROWS * EMBED_DIM
NUM_CORES = 2
NUM_SUBCORES = 16
NUM_WORKERS = NUM_CORES * NUM_SUBCORES
B_PER_W = BATCH // NUM_WORKERS
LANES = 16
N_GROUPS = B_PER_W // LANES
ELEMS_PER_W = B_PER_W * EMBED_DIM
CHUNK = 128
N_CHUNKS = ELEMS_PER_W // CHUNK
FIRE = 16

_mesh = plsc.VectorSubcoreMesh(core_axis_name="c", subcore_axis_name="s")


@functools.partial(
    pl.kernel,
    mesh=_mesh,
    compiler_params=pltpu.CompilerParams(use_tc_tiling_on_sc=False),
    out_type=jax.ShapeDtypeStruct((BATCH * EMBED_DIM,), jnp.float32),
    scratch_types=[
        pltpu.VMEM((B_PER_W,), jnp.int32),
        pltpu.VMEM((ELEMS_PER_W,), jnp.int32),
        pltpu.VMEM((ELEMS_PER_W,), jnp.float32),
        pltpu.SemaphoreType.DMA,
    ],
)
def _gather_kernel(tab_hbm, idx_hbm, out_hbm, idx_v, pos_v, val_v, sem):
    wid = lax.axis_index("s") * NUM_CORES + lax.axis_index("c")
    base = wid * B_PER_W
    pltpu.sync_copy(idx_hbm.at[pl.ds(base, B_PER_W)], idx_v)
    for g in range(N_GROUPS):
        v = idx_v[pl.ds(g * LANES, LANES)]
        for r in range(EMBED_DIM):
            pos_v[pl.ds(g * (EMBED_DIM * LANES) + r * LANES, LANES)] = (
                v + r * NUM_ROWS
            )
    for b in range(N_CHUNKS // FIRE):
        copies = []
        for k in range(FIRE):
            c0 = (b * FIRE + k) * CHUNK
            copies.append(
                pltpu.async_copy(
                    tab_hbm.at[pos_v.at[pl.ds(c0, CHUNK)]],
                    val_v.at[pl.ds(c0, CHUNK)],
                    sem,
                )
            )
        for c in copies:
            c.wait()
    pltpu.sync_copy(val_v, out_hbm.at[pl.ds(wid * ELEMS_PER_W, ELEMS_PER_W)])


def kernel(indices, table):
    tflat = table.T.reshape(FLAT)
    idx = indices.astype(jnp.int32)
    out = _gather_kernel(tflat, idx)
    return (
        out.reshape(BATCH // LANES, EMBED_DIM, LANES)
        .transpose(0, 2, 1)
        .reshape(BATCH, EMBED_DIM)
    )


# speedup vs baseline: 1.2100x; 1.1784x over previous
import functools

import jax
import jax.numpy as jnp
from jax import lax
from jax.experimental import pallas as pl
from jax.experimental.pallas import tpu as pltpu
from jax.experimental.pallas import tpu_sc as plsc

EMBED_DIM = 32
BATCH = 16384
NUM_ROWS = 100001
PAD_ROWS = 100008
PADW = 128
NUM_CORES = 2
NUM_SUBCORES = 16
NUM_WORKERS = NUM_CORES * NUM_SUBCORES
B_PER_W = BATCH // NUM_WORKERS
CHUNK = 128
N_CHUNKS = B_PER_W // CHUNK

_mesh = plsc.VectorSubcoreMesh(core_axis_name="c", subcore_axis_name="s")


@functools.partial(
    pl.kernel,
    mesh=_mesh,
    out_type=jax.ShapeDtypeStruct((BATCH // CHUNK, CHUNK, PADW), jnp.float32),
    scratch_types=[
        pltpu.VMEM((N_CHUNKS, CHUNK), jnp.int32),
        pltpu.VMEM((N_CHUNKS, CHUNK, PADW), jnp.float32),
        pltpu.SemaphoreType.DMA,
        pltpu.SemaphoreType.DMA,
        pltpu.SemaphoreType.DMA,
        pltpu.SemaphoreType.DMA,
        pltpu.SemaphoreType.DMA,
    ],
)
def _gather_kernel(tab_hbm, idx_hbm, out_hbm, idx_v, rows_v, s0, s1, s2, s3, sw):
    sems = (s0, s1, s2, s3)
    wid = lax.axis_index("s") * NUM_CORES + lax.axis_index("c")
    base = wid * N_CHUNKS
    pltpu.sync_copy(idx_hbm.at[pl.ds(base, N_CHUNKS)], idx_v)
    gathers = [
        pltpu.async_copy(tab_hbm.at[idx_v.at[j]], rows_v.at[j], sems[j])
        for j in range(N_CHUNKS)
    ]
    writes = []
    for j in range(N_CHUNKS):
        gathers[j].wait()
        writes.append(
            pltpu.async_copy(rows_v.at[j], out_hbm.at[base + j], sw)
        )
    for w in writes:
        w.wait()


def kernel(indices, table):
    tab_pad = jnp.pad(table, ((0, PAD_ROWS - NUM_ROWS), (0, PADW - EMBED_DIM)))
    idx = indices.astype(jnp.int32).reshape(BATCH // CHUNK, CHUNK)
    out = _gather_kernel(tab_pad, idx)
    return out.reshape(BATCH, PADW)[:, :EMBED_DIM]
